# baseline (speedup 1.0000x reference)
import functools

import jax
import jax.numpy as jnp
from jax import lax
from jax.experimental import pallas as pl
from jax.experimental.pallas import tpu as pltpu
from jax.experimental.pallas import tpu_sc as plsc

B = 16384
D = 64
NC = 2
NS = 16
NW = NC * NS

SB = 8192
BPW = SB // NW
CHUNK = 2048


def _sc_gather(uidx, iidx, eug, eig, eum, eim):
    mesh = plsc.VectorSubcoreMesh(core_axis_name="c", subcore_axis_name="s")
    out_type = tuple(
        jax.ShapeDtypeStruct((SB, D), jnp.float32) for _ in range(4)
    )
    CH = max(d for d in (256, 192, 128, 64, 32, 16) if BPW % d == 0)
    scratch_types = [
        pltpu.VMEM((BPW,), jnp.int32),
        pltpu.VMEM((CH, D), jnp.float32),
        pltpu.VMEM((CH, D), jnp.float32),
        pltpu.SemaphoreType.DMA,
        pltpu.SemaphoreType.DMA,
    ]

    def body(u_hbm, i_hbm, t0, t1, t2, t3, o0, o1, o2, o3,
             idx_v, buf0, buf1, sem0, sem1):
        wid = lax.axis_index("s") * NC + lax.axis_index("c")
        base = wid * BPW

        def pass_over(idx_hbm, ta, tb, oa, ob):
            pltpu.sync_copy(idx_hbm.at[pl.ds(base, BPW)], idx_v)
            for c in range(BPW // CH):
                off = c * CH

                def grp(g, _):
                    r0 = g * 16
                    v = idx_v[pl.ds(off + r0, 16)]
                    for j in range(16):
                        s = v[j]
                        pltpu.async_copy(
                            ta.at[pl.ds(s, 1)], buf0.at[pl.ds(r0 + j, 1)],
                            sem0)
                        pltpu.async_copy(
                            tb.at[pl.ds(s, 1)], buf1.at[pl.ds(r0 + j, 1)],
                            sem1)
                    return _

                lax.fori_loop(0, CH // 16, grp, 0)
                pltpu.make_async_copy(ta.at[pl.ds(0, CH)], buf0, sem0).wait()
                pltpu.make_async_copy(tb.at[pl.ds(0, CH)], buf1, sem1).wait()
                pltpu.sync_copy(buf0, oa.at[pl.ds(base + off, CH)])
                pltpu.sync_copy(buf1, ob.at[pl.ds(base + off, CH)])

        pass_over(u_hbm, t0, t2, o0, o2)
        pass_over(i_hbm, t1, t3, o1, o3)

    return pl.kernel(
        body, out_type=out_type, mesh=mesh, scratch_types=scratch_types,
        compiler_params=pltpu.CompilerParams(use_tc_tiling_on_sc=True),
    )(uidx, iidx, eug, eig, eum, eim)


def _dense(gu, gi, mu, mi, au, ai, b12, kg, k3, m1p, m2t, m2b, m3r, m3b):
    f32 = jnp.float32
    p = gu * gi
    h2 = jnp.maximum(
        jnp.dot(mu, au, preferred_element_type=f32)
        + jnp.dot(mi, ai, preferred_element_type=f32)
        + b12, 0.0)
    z1 = jnp.maximum(
        jnp.dot(p, kg, preferred_element_type=f32)
        + jnp.dot(h2, k3, preferred_element_type=f32)
        + m1p, 0.0)
    z2 = jnp.maximum(
        jnp.dot(z1, m2t, preferred_element_type=f32) + m2b, 0.0)
    s = jnp.sum(z2 * m3r, axis=1) + m3b[0, 0]
    return 1.0 / (1.0 + jnp.exp(-s))


def _tc_dense_body(gu, gi, mu, mi, au, ai, b12, kg, k3, m1p, m2t, m2b, m3r,
                   m3b, out):
    out[...] = _dense(
        gu[...], gi[...], mu[...], mi[...], au[...], ai[...], b12[...],
        kg[...], k3[...], m1p[...], m2t[...], m2b[...], m3r[...], m3b[...])


def _tc_gather_dense_body(us, isx, t0, t1, t2, t3, au, ai, b12, kg, k3, m1p,
                          m2t, m2b, m3r, m3b, out, gu, gi, mu, mi, sem0,
                          sem1, sem2, sem3):
    def row(r, _):
        su = us[r]
        si = isx[r]
        pltpu.async_copy(t0.at[pl.ds(su, 1)], gu.at[pl.ds(r, 1)], sem0)
        pltpu.async_copy(t1.at[pl.ds(si, 1)], gi.at[pl.ds(r, 1)], sem1)
        pltpu.async_copy(t2.at[pl.ds(su, 1)], mu.at[pl.ds(r, 1)], sem2)
        pltpu.async_copy(t3.at[pl.ds(si, 1)], mi.at[pl.ds(r, 1)], sem3)
        return _

    lax.fori_loop(0, CHUNK, row, 0)
    pltpu.make_async_copy(t0.at[pl.ds(0, CHUNK)], gu, sem0).wait()
    pltpu.make_async_copy(t1.at[pl.ds(0, CHUNK)], gi, sem1).wait()
    pltpu.make_async_copy(t2.at[pl.ds(0, CHUNK)], mu, sem2).wait()
    pltpu.make_async_copy(t3.at[pl.ds(0, CHUNK)], mi, sem3).wait()
    out[...] = _dense(
        gu[...], gi[...], mu[...], mi[...], au[...], ai[...], b12[...],
        kg[...], k3[...], m1p[...], m2t[...], m2b[...], m3r[...], m3b[...])


def _fold_weights(Wg, W1, b1, W2, b2, W3, b3, M1, m1, M2, m2, M3, m3):
    W12 = W2 @ W1
    b12 = (W2 @ b1 + b2)[None, :]
    Au = W12[:, :D].T
    Ai = W12[:, D:].T
    Kg = (M1[:, :8] @ Wg).T
    K3 = (M1[:, 8:] @ W3).T
    m1p = (m1 + M1[:, 8:] @ b3)[None, :]
    return Au, Ai, b12, Kg, K3, m1p, M2.T, m2[None, :], M3, m3[None, :]


def kernel(user_indices, item_indices, Eug, Eig, Eum, Eim, Wg, W1, b1, W2,
           b2, W3, b3, M1, m1, M2, m2, M3, m3):
    uidx = user_indices.astype(jnp.int32)
    iidx = item_indices.astype(jnp.int32)

    folded = _fold_weights(Wg, W1, b1, W2, b2, W3, b3, M1, m1, M2, m2, M3,
                           m3)
    wspecs = [
        pl.BlockSpec(s, lambda i: tuple(0 for _ in s))
        for s in [(D, D), (D, D), (1, D), (D, 16), (D, 16), (1, 16),
                  (16, 8), (1, 8), (1, 8), (1, 1)]
    ]

    gu, gi, mu, mi = _sc_gather(uidx[:SB], iidx[:SB], Eug, Eig, Eum, Eim)

    BT = B - SB
    if BT:
        any_spec = pl.BlockSpec(memory_space=pltpu.MemorySpace.HBM)
        idx_spec = pl.BlockSpec((CHUNK,), lambda i: (i,),
                                memory_space=pltpu.MemorySpace.SMEM)
        out_tc = pl.pallas_call(
            _tc_gather_dense_body,
            grid=(BT // CHUNK,),
            in_specs=[idx_spec, idx_spec] + [any_spec] * 4 + wspecs,
            out_specs=pl.BlockSpec((CHUNK,), lambda i: (i,)),
            out_shape=jax.ShapeDtypeStruct((BT,), jnp.float32),
            scratch_shapes=[pltpu.VMEM((CHUNK, D), jnp.float32)] * 4
            + [pltpu.SemaphoreType.DMA] * 4,
        )(uidx[SB:], iidx[SB:], Eug, Eig, Eum, Eim, *folded)

    data_spec = pl.BlockSpec((CHUNK, D), lambda i: (i, 0))
    out_sc = pl.pallas_call(
        _tc_dense_body,
        grid=(SB // CHUNK,),
        in_specs=[data_spec] * 4 + wspecs,
        out_specs=pl.BlockSpec((CHUNK,), lambda i: (i,)),
        out_shape=jax.ShapeDtypeStruct((SB,), jnp.float32),
    )(gu, gi, mu, mi, *folded)

    if BT:
        return jnp.concatenate([out_sc, out_tc])
    return out_sc


# speedup vs baseline: 1.0803x; 1.0803x over previous
import functools

import jax
import jax.numpy as jnp
from jax import lax
from jax.experimental import pallas as pl
from jax.experimental.pallas import tpu as pltpu
from jax.experimental.pallas import tpu_sc as plsc

B = 16384
D = 64
NC = 2
NS = 16
NW = NC * NS

SB = 16384
BPW = SB // NW
CHUNK = 2048


def _sc_gather(uidx, iidx, eug, eig, eum, eim):
    mesh = plsc.VectorSubcoreMesh(core_axis_name="c", subcore_axis_name="s")
    out_type = tuple(
        jax.ShapeDtypeStruct((SB, D), jnp.float32) for _ in range(4)
    )
    CH = max(d for d in (256, 192, 128, 64, 32, 16) if BPW % d == 0)
    scratch_types = [
        pltpu.VMEM((BPW,), jnp.int32),
        pltpu.VMEM((CH, D), jnp.float32),
        pltpu.VMEM((CH, D), jnp.float32),
        pltpu.SemaphoreType.DMA,
        pltpu.SemaphoreType.DMA,
    ]

    def body(u_hbm, i_hbm, t0, t1, t2, t3, o0, o1, o2, o3,
             idx_v, buf0, buf1, sem0, sem1):
        wid = lax.axis_index("s") * NC + lax.axis_index("c")
        base = wid * BPW

        def pass_over(idx_hbm, ta, tb, oa, ob):
            pltpu.sync_copy(idx_hbm.at[pl.ds(base, BPW)], idx_v)
            for c in range(BPW // CH):
                off = c * CH

                def grp(g, _):
                    r0 = g * 16
                    v = idx_v[pl.ds(off + r0, 16)]
                    for j in range(16):
                        s = v[j]
                        pltpu.async_copy(
                            ta.at[pl.ds(s, 1)], buf0.at[pl.ds(r0 + j, 1)],
                            sem0)
                        pltpu.async_copy(
                            tb.at[pl.ds(s, 1)], buf1.at[pl.ds(r0 + j, 1)],
                            sem1)
                    return _

                lax.fori_loop(0, CH // 16, grp, 0)
                pltpu.make_async_copy(ta.at[pl.ds(0, CH)], buf0, sem0).wait()
                pltpu.make_async_copy(tb.at[pl.ds(0, CH)], buf1, sem1).wait()
                pltpu.sync_copy(buf0, oa.at[pl.ds(base + off, CH)])
                pltpu.sync_copy(buf1, ob.at[pl.ds(base + off, CH)])

        pass_over(u_hbm, t0, t2, o0, o2)
        pass_over(i_hbm, t1, t3, o1, o3)

    return pl.kernel(
        body, out_type=out_type, mesh=mesh, scratch_types=scratch_types,
        compiler_params=pltpu.CompilerParams(use_tc_tiling_on_sc=True),
    )(uidx, iidx, eug, eig, eum, eim)


def _dense(gu, gi, mu, mi, au, ai, b12, kg, k3, m1p, m2t, m2b, m3r, m3b):
    f32 = jnp.float32
    p = gu * gi
    h2 = jnp.maximum(
        jnp.dot(mu, au, preferred_element_type=f32)
        + jnp.dot(mi, ai, preferred_element_type=f32)
        + b12, 0.0)
    z1 = jnp.maximum(
        jnp.dot(p, kg, preferred_element_type=f32)
        + jnp.dot(h2, k3, preferred_element_type=f32)
        + m1p, 0.0)
    z2 = jnp.maximum(
        jnp.dot(z1, m2t, preferred_element_type=f32) + m2b, 0.0)
    s = jnp.sum(z2 * m3r, axis=1) + m3b[0, 0]
    return 1.0 / (1.0 + jnp.exp(-s))


def _tc_dense_body(gu, gi, mu, mi, au, ai, b12, kg, k3, m1p, m2t, m2b, m3r,
                   m3b, out):
    out[...] = _dense(
        gu[...], gi[...], mu[...], mi[...], au[...], ai[...], b12[...],
        kg[...], k3[...], m1p[...], m2t[...], m2b[...], m3r[...], m3b[...])


def _tc_gather_dense_body(us, isx, t0, t1, t2, t3, au, ai, b12, kg, k3, m1p,
                          m2t, m2b, m3r, m3b, out, gu, gi, mu, mi, sem0,
                          sem1, sem2, sem3):
    def row(r, _):
        su = us[r]
        si = isx[r]
        pltpu.async_copy(t0.at[pl.ds(su, 1)], gu.at[pl.ds(r, 1)], sem0)
        pltpu.async_copy(t1.at[pl.ds(si, 1)], gi.at[pl.ds(r, 1)], sem1)
        pltpu.async_copy(t2.at[pl.ds(su, 1)], mu.at[pl.ds(r, 1)], sem2)
        pltpu.async_copy(t3.at[pl.ds(si, 1)], mi.at[pl.ds(r, 1)], sem3)
        return _

    lax.fori_loop(0, CHUNK, row, 0)
    pltpu.make_async_copy(t0.at[pl.ds(0, CHUNK)], gu, sem0).wait()
    pltpu.make_async_copy(t1.at[pl.ds(0, CHUNK)], gi, sem1).wait()
    pltpu.make_async_copy(t2.at[pl.ds(0, CHUNK)], mu, sem2).wait()
    pltpu.make_async_copy(t3.at[pl.ds(0, CHUNK)], mi, sem3).wait()
    out[...] = _dense(
        gu[...], gi[...], mu[...], mi[...], au[...], ai[...], b12[...],
        kg[...], k3[...], m1p[...], m2t[...], m2b[...], m3r[...], m3b[...])


def _fold_weights(Wg, W1, b1, W2, b2, W3, b3, M1, m1, M2, m2, M3, m3):
    W12 = W2 @ W1
    b12 = (W2 @ b1 + b2)[None, :]
    Au = W12[:, :D].T
    Ai = W12[:, D:].T
    Kg = (M1[:, :8] @ Wg).T
    K3 = (M1[:, 8:] @ W3).T
    m1p = (m1 + M1[:, 8:] @ b3)[None, :]
    return Au, Ai, b12, Kg, K3, m1p, M2.T, m2[None, :], M3, m3[None, :]


def kernel(user_indices, item_indices, Eug, Eig, Eum, Eim, Wg, W1, b1, W2,
           b2, W3, b3, M1, m1, M2, m2, M3, m3):
    uidx = user_indices.astype(jnp.int32)
    iidx = item_indices.astype(jnp.int32)

    folded = _fold_weights(Wg, W1, b1, W2, b2, W3, b3, M1, m1, M2, m2, M3,
                           m3)
    wspecs = [
        pl.BlockSpec(s, lambda i: tuple(0 for _ in s))
        for s in [(D, D), (D, D), (1, D), (D, 16), (D, 16), (1, 16),
                  (16, 8), (1, 8), (1, 8), (1, 1)]
    ]

    gu, gi, mu, mi = _sc_gather(uidx[:SB], iidx[:SB], Eug, Eig, Eum, Eim)

    BT = B - SB
    if BT:
        any_spec = pl.BlockSpec(memory_space=pltpu.MemorySpace.HBM)
        idx_spec = pl.BlockSpec((CHUNK,), lambda i: (i,),
                                memory_space=pltpu.MemorySpace.SMEM)
        out_tc = pl.pallas_call(
            _tc_gather_dense_body,
            grid=(BT // CHUNK,),
            in_specs=[idx_spec, idx_spec] + [any_spec] * 4 + wspecs,
            out_specs=pl.BlockSpec((CHUNK,), lambda i: (i,)),
            out_shape=jax.ShapeDtypeStruct((BT,), jnp.float32),
            scratch_shapes=[pltpu.VMEM((CHUNK, D), jnp.float32)] * 4
            + [pltpu.SemaphoreType.DMA] * 4,
        )(uidx[SB:], iidx[SB:], Eug, Eig, Eum, Eim, *folded)

    data_spec = pl.BlockSpec((CHUNK, D), lambda i: (i, 0))
    out_sc = pl.pallas_call(
        _tc_dense_body,
        grid=(SB // CHUNK,),
        in_specs=[data_spec] * 4 + wspecs,
        out_specs=pl.BlockSpec((CHUNK,), lambda i: (i,)),
        out_shape=jax.ShapeDtypeStruct((SB,), jnp.float32),
    )(gu, gi, mu, mi, *folded)

    if BT:
        return jnp.concatenate([out_sc, out_tc])
    return out_sc
